# baseline (speedup 1.0000x reference)
import dataclasses
import functools

import jax
import jax.numpy as jnp
from jax import lax
from jax.experimental import pallas as pl
from jax.experimental.pallas import tpu as pltpu
from jax.experimental.pallas import tpu_sc as plsc

N = 10000
E = 160000
D_IN = 128
HID = 64
OUT = 7
K = 7

NP_ = 10240
EP_ = 163840
NWORK = 32
CHUNK = 128
NCHTOT = EP_ // CHUNK
FAST_CORE = 1
NCH1_F, NCH1_S = 40, 40
NCH3_F, NCH3_S = 40, 40
ESZ = 10016
TSLICE = NP_ // 16
NB = 512
TW = 80

def _sc_compiler_params():
    cp = pltpu.CompilerParams()
    if "needs_layout_passes" in pltpu.CompilerParams.__dataclass_fields__:
        cp = dataclasses.replace(cp, needs_layout_passes=False)
    cp = dataclasses.replace(cp, use_tc_tiling_on_sc=False)
    return cp

GRID = NP_ // NB
H2W = 16


def _p0_body(x_ref, w1_ref, b1_ref, a1s_ref, a1d_ref, cb_ref,
             h1_ref, es_ref, ed_ref, lps_ref):
    xb = x_ref[...]
    h1 = jnp.dot(xb, w1_ref[...], preferred_element_type=jnp.float32)
    h1 = h1 + b1_ref[...]
    h1_ref[...] = jnp.concatenate(
        [h1, jnp.ones((NB, 1), jnp.float32),
         jnp.zeros((NB, TW - HID - 1), jnp.float32)], axis=1)
    es_ref[0, :] = jnp.dot(h1, a1s_ref[...], preferred_element_type=jnp.float32)[:, 0]
    ed_ref[0, :] = jnp.dot(h1, a1d_ref[...], preferred_element_type=jnp.float32)[:, 0]
    ssum = jnp.zeros((NB,), jnp.float32)
    slog = jnp.zeros((NB,), jnp.float32)
    for k in range(K):
        d = xb - cb_ref[k, :][None, :]
        dist = jnp.sqrt(jnp.sum(d * d, axis=1) + 1e-12)
        sc = jnp.exp(-dist) + 1e-10
        ssum = ssum + sc
        slog = slog + jnp.log(sc)
    lps_ref[0, :] = slog - K * jnp.log(ssum)


def _run_p0(x_p, W1, b1r, a1s, a1d, C_b):
    return pl.pallas_call(
        _p0_body,
        grid=(GRID,),
        in_specs=[
            pl.BlockSpec((NB, D_IN), lambda j: (j, 0)),
            pl.BlockSpec((D_IN, HID), lambda j: (0, 0)),
            pl.BlockSpec((1, HID), lambda j: (0, 0)),
            pl.BlockSpec((HID, 1), lambda j: (0, 0)),
            pl.BlockSpec((HID, 1), lambda j: (0, 0)),
            pl.BlockSpec((K, D_IN), lambda j: (0, 0)),
        ],
        out_specs=[
            pl.BlockSpec((NB, TW), lambda j: (j, 0)),
            pl.BlockSpec((1, NB), lambda j: (0, j)),
            pl.BlockSpec((1, NB), lambda j: (0, j)),
            pl.BlockSpec((1, NB), lambda j: (0, j)),
        ],
        out_shape=[
            jax.ShapeDtypeStruct((NP_, TW), jnp.float32),
            jax.ShapeDtypeStruct((1, NP_), jnp.float32),
            jax.ShapeDtypeStruct((1, NP_), jnp.float32),
            jax.ShapeDtypeStruct((1, NP_), jnp.float32),
        ],
    )(x_p, W1, b1r, a1s, a1d, C_b)


def _sc1_body(es_h, ed_h, src_h, dst_h, h1_h,
              ex_h, acc_h,
              es_v, ed_v, srcv, dstv, exv, rows_a, rows_b,
              r80_a, r80_b,
              gsa, gsb, ssa, ssb,
              acc_sh):
    c = lax.axis_index("core")
    s = lax.axis_index("subcore")
    wid = c * 16 + s

    z16v = jnp.zeros((16,), jnp.float32)

    @pl.loop(0, CHUNK)
    def _zr(r):
        for q in range(TW // 16):
            r80_a[r, pl.ds(q * 16, 16)] = z16v

    for q in range(TSLICE // CHUNK):
        pltpu.sync_copy(r80_a, acc_sh.at[pl.ds(s * TSLICE + q * CHUNK, CHUNK)])
    pltpu.sync_copy(es_h.at[pl.ds(0, ESZ)], es_v)
    pltpu.sync_copy(ed_h.at[pl.ds(0, ESZ)], ed_v)

    def stage(nch, base):
        pltpu.sync_copy(src_h.at[pl.ds(base, nch)], srcv.at[pl.ds(0, nch)])
        pltpu.sync_copy(dst_h.at[pl.ds(base, nch)], dstv.at[pl.ds(0, nch)])

    @pl.when(c == FAST_CORE)
    def _():
        stage(NCH1_F, s * NCH1_F)

    @pl.when(c != FAST_CORE)
    def _():
        stage(NCH1_S, 16 * NCH1_F + s * NCH1_S)
    plsc.subcore_barrier()

    def phase1(nch):
        @pl.loop(0, nch)
        def _exch(ch):
            @pl.loop(0, CHUNK // 16)
            def _vec(i):
                s16 = srcv[ch, pl.ds(i * 16, 16)]
                d16 = dstv[ch, pl.ds(i * 16, 16)]
                ev = plsc.load_gather(es_v, [s16])
                dv = plsc.load_gather(ed_v, [d16])
                e = ev + dv
                e = jnp.where(e > 0, e, 0.2 * e)
                exv[ch, pl.ds(i * 16, 16)] = jnp.exp(e)

    def g_start(ch, buf, sem):
        pltpu.make_async_copy(h1_h.at[srcv.at[ch]], buf, sem).start()

    def g_wait(ch, buf, sem):
        pltpu.make_async_copy(h1_h.at[srcv.at[ch]], buf, sem).wait()

    def s_start(ch, obuf, sem):
        pltpu.make_async_copy(obuf, acc_sh.at[dstv.at[ch]], sem).start(add=True)

    def s_wait(ch, obuf, sem):
        pltpu.make_async_copy(obuf, acc_sh.at[dstv.at[ch]], sem).wait()

    def scale(ch, buf, obuf):
        @pl.loop(0, CHUNK // 16)
        def _scale(i):
            exvec = exv[ch, pl.ds(i * 16, 16)]
            for l in range(16):
                j = i * 16 + l
                sc = exvec[l]
                for q in range(TW // 16):
                    obuf[j, pl.ds(q * 16, 16)] = buf[j, pl.ds(q * 16, 16)] * sc

    def pipeline(nch):
        phase1(nch)
        g_start(0, rows_a, gsa)
        g_start(1, rows_b, gsb)

        @pl.loop(0, nch // 2)
        def _pipe(i):
            ch0 = i * 2
            ch1 = ch0 + 1
            g_wait(ch0, rows_a, gsa)

            @pl.when(i > 0)
            def _():
                s_wait(ch0, r80_a, ssa)
            scale(ch0, rows_a, r80_a)
            s_start(ch0, r80_a, ssa)

            @pl.when(i < nch // 2 - 1)
            def _():
                g_start(ch0 + 2, rows_a, gsa)

            g_wait(ch1, rows_b, gsb)

            @pl.when(i > 0)
            def _():
                s_wait(ch1, r80_b, ssb)
            scale(ch1, rows_b, r80_b)
            s_start(ch1, r80_b, ssb)

            @pl.when(i < nch // 2 - 1)
            def _():
                g_start(ch1 + 2, rows_b, gsb)

        s_wait(0, r80_a, ssa)
        s_wait(1, r80_b, ssb)

    @pl.when(c == FAST_CORE)
    def _():
        pipeline(NCH1_F)
        pltpu.sync_copy(exv.at[pl.ds(0, NCH1_F)],
                        ex_h.at[pl.ds(s * NCH1_F, NCH1_F)])

    @pl.when(c != FAST_CORE)
    def _():
        pipeline(NCH1_S)
        pltpu.sync_copy(exv.at[pl.ds(0, NCH1_S)],
                        ex_h.at[pl.ds(16 * NCH1_F + s * NCH1_S, NCH1_S)])
    plsc.subcore_barrier()
    for q in range(TSLICE // CHUNK):
        buf = r80_a if q % 2 == 0 else r80_b
        pltpu.sync_copy(acc_sh.at[pl.ds(s * TSLICE + q * CHUNK, CHUNK)], buf)
        pltpu.sync_copy(buf, acc_h.at[wid * (TSLICE // CHUNK) + q])


def _run_sc1(es, ed, src3, dst3, h1p):
    kern = pl.kernel(
        _sc1_body,
        out_type=[
            jax.ShapeDtypeStruct((NCHTOT, CHUNK), jnp.float32),
            jax.ShapeDtypeStruct((NWORK * TSLICE // CHUNK, CHUNK, TW),
                                 jnp.float32),
        ],
        mesh=plsc.VectorSubcoreMesh(core_axis_name="core",
                                    subcore_axis_name="subcore"),
        scratch_types=[
            pltpu.VMEM((ESZ,), jnp.float32),
            pltpu.VMEM((ESZ,), jnp.float32),
            pltpu.VMEM((NCH1_F, CHUNK), jnp.int32),
            pltpu.VMEM((NCH1_F, CHUNK), jnp.int32),
            pltpu.VMEM((NCH1_F, CHUNK), jnp.float32),
            pltpu.VMEM((CHUNK, TW), jnp.float32),
            pltpu.VMEM((CHUNK, TW), jnp.float32),
            pltpu.VMEM((CHUNK, TW), jnp.float32),
            pltpu.VMEM((CHUNK, TW), jnp.float32),
            pltpu.SemaphoreType.DMA,
            pltpu.SemaphoreType.DMA,
            pltpu.SemaphoreType.DMA,
            pltpu.SemaphoreType.DMA,
            pltpu.VMEM_SHARED((NP_, TW), jnp.float32),
        ],
        compiler_params=_sc_compiler_params(),
    )
    return kern(es, ed, src3, dst3, h1p)


def _p2a_body(acc_ref, y_ref, w2_ref, b2_ref,
              z0_ref, h2_ref, rden_ref, ca_ref, sums_ref, cnt_ref):
    j = pl.program_id(0)

    @pl.when(j == 0)
    def _():
        sums_ref[...] = jnp.zeros_like(sums_ref)
        cnt_ref[...] = jnp.zeros_like(cnt_ref)

    a = acc_ref[0] + acc_ref[1]
    rden = 1.0 / (a[:, HID] + 1e-16)
    rden_ref[0, :] = rden
    z0 = a[:, :HID] * rden[:, None]
    z0_ref[...] = z0
    x1 = jnp.where(z0 > 0, z0, jnp.exp(jnp.minimum(z0, 0.0)) - 1.0)
    h2_ref[...] = jnp.dot(x1, w2_ref[...], preferred_element_type=jnp.float32) + b2_ref[...]
    yb = y_ref[0, :]
    y1h = (lax.broadcasted_iota(jnp.int32, (NB, 8), 1) == yb[:, None]).astype(jnp.float32)
    sums_ref[...] += jax.lax.dot_general(
        y1h, z0, (((0,), (0,)), ((), ())), preferred_element_type=jnp.float32)
    cnt_ref[...] += jnp.sum(y1h, axis=0, keepdims=True)
    ca_ref[...] = sums_ref[...] / jnp.maximum(cnt_ref[0, :], 1.0)[:, None]


def _run_p2a(acc2, y2, W2p, b2p):
    return pl.pallas_call(
        _p2a_body,
        grid=(GRID,),
        in_specs=[
            pl.BlockSpec((2, NB, TW), lambda j: (0, j, 0)),
            pl.BlockSpec((1, NB), lambda j: (0, j)),
            pl.BlockSpec((HID, H2W), lambda j: (0, 0)),
            pl.BlockSpec((1, H2W), lambda j: (0, 0)),
        ],
        out_specs=[
            pl.BlockSpec((NB, HID), lambda j: (j, 0)),
            pl.BlockSpec((NB, H2W), lambda j: (j, 0)),
            pl.BlockSpec((1, NB), lambda j: (0, j)),
            pl.BlockSpec((8, HID), lambda j: (0, 0)),
        ],
        out_shape=[
            jax.ShapeDtypeStruct((NP_, HID), jnp.float32),
            jax.ShapeDtypeStruct((NP_, H2W), jnp.float32),
            jax.ShapeDtypeStruct((1, NP_), jnp.float32),
            jax.ShapeDtypeStruct((8, HID), jnp.float32),
        ],
        scratch_shapes=[
            pltpu.VMEM((8, HID), jnp.float32),
            pltpu.VMEM((1, 8), jnp.float32),
        ],
    )(acc2, y2, W2p, b2p)


def _p2b_body(z0_ref, ca_ref, y_ref, lps_ref, q_ref, h2_ref, phi_ref, v_ref):
    zb = z0_ref[...]
    dists = []
    for k in range(K):
        dk = zb - ca_ref[k, :][None, :]
        dists.append(jnp.sqrt(jnp.sum(dk * dk, axis=1) + 1e-12))
    dist = jnp.stack(dists, axis=1)
    sc = jnp.exp(-dist) + 1e-10
    phi = sc / jnp.sum(sc, axis=1, keepdims=True)
    phi_ref[...] = phi
    yb = y_ref[0, :]
    y1h = (lax.broadcasted_iota(jnp.int32, (NB, 8), 1) == yb[:, None]).astype(jnp.float32)
    lq = jnp.log(q_ref[...])
    lq8 = jnp.concatenate([lq, jnp.zeros((1, K), jnp.float32)], axis=0)
    diff_b = phi * lps_ref[0, :][:, None] - jnp.dot(
        y1h, lq8, preferred_element_type=jnp.float32)
    dp = diff_b * phi
    dpsum = jnp.sum(dp, axis=1, keepdims=True)
    w = jnp.zeros((NB, HID), jnp.float32)
    t = jnp.zeros((NB, HID), jnp.float32)
    for k in range(K):
        dk = zb - ca_ref[k, :][None, :]
        dn = dk / dist[:, k][:, None]
        w = w + phi[:, k][:, None] * dn
        t = t + dp[:, k][:, None] * dn
    v = -(t - dpsum * w)
    v_ref[...] = jnp.concatenate([v, h2_ref[...]], axis=1)


def _run_p2b(z0p, ca, y2, lps, Q, h2p):
    return pl.pallas_call(
        _p2b_body,
        grid=(GRID,),
        in_specs=[
            pl.BlockSpec((NB, HID), lambda j: (j, 0)),
            pl.BlockSpec((8, HID), lambda j: (0, 0)),
            pl.BlockSpec((1, NB), lambda j: (0, j)),
            pl.BlockSpec((1, NB), lambda j: (0, j)),
            pl.BlockSpec((K, K), lambda j: (0, 0)),
            pl.BlockSpec((NB, H2W), lambda j: (j, 0)),
        ],
        out_specs=[
            pl.BlockSpec((NB, K), lambda j: (j, 0)),
            pl.BlockSpec((NB, TW), lambda j: (j, 0)),
        ],
        out_shape=[
            jax.ShapeDtypeStruct((NP_, K), jnp.float32),
            jax.ShapeDtypeStruct((NP_, TW), jnp.float32),
        ],
    )(z0p, ca, y2, lps, Q, h2p)


def _sc3_body(src_h, dst_h, ex_h, rden_h, vh2_h, h1_h,
              o2_h,
              rden_v, srcv, dstv, exv, v_a, v_b, h_a, h_b, o_a, o_b,
              gva, gvb, gha, ghb, osa, osb,
              o2_sh):
    c = lax.axis_index("core")
    s = lax.axis_index("subcore")
    wid = c * 16 + s

    z16v = jnp.zeros((16,), jnp.float32)

    @pl.loop(0, CHUNK)
    def _zr(r):
        o_a[r, pl.ds(0, H2W)] = z16v

    for q in range(TSLICE // CHUNK):
        pltpu.sync_copy(o_a, o2_sh.at[pl.ds(s * TSLICE + q * CHUNK, CHUNK)])
    pltpu.sync_copy(rden_h.at[pl.ds(0, ESZ)], rden_v)

    def stage(nch, base):
        pltpu.sync_copy(src_h.at[pl.ds(base, nch)], srcv.at[pl.ds(0, nch)])
        pltpu.sync_copy(dst_h.at[pl.ds(base, nch)], dstv.at[pl.ds(0, nch)])
        pltpu.sync_copy(ex_h.at[pl.ds(base, nch)], exv.at[pl.ds(0, nch)])

    @pl.when(c == FAST_CORE)
    def _():
        stage(NCH3_F, s * NCH3_F)

    @pl.when(c != FAST_CORE)
    def _():
        stage(NCH3_S, 16 * NCH3_F + s * NCH3_S)
    plsc.subcore_barrier()

    inv_n = jnp.float32(1.0 / N)

    def g_start(ch, vbuf, hbuf, vsem, hsem):
        pltpu.make_async_copy(vh2_h.at[srcv.at[ch]], vbuf, vsem).start()
        pltpu.make_async_copy(h1_h.at[dstv.at[ch]], hbuf, hsem).start()

    def g_wait(ch, vbuf, hbuf, vsem, hsem):
        pltpu.make_async_copy(vh2_h.at[srcv.at[ch]], vbuf, vsem).wait()
        pltpu.make_async_copy(h1_h.at[dstv.at[ch]], hbuf, hsem).wait()

    def s_start(ch, obuf, sem):
        pltpu.make_async_copy(obuf, o2_sh.at[dstv.at[ch]], sem).start(add=True)

    def s_wait(ch, obuf, sem):
        pltpu.make_async_copy(obuf, o2_sh.at[dstv.at[ch]], sem).wait()

    def compute(ch, vrows, hrows, obuf):
        @pl.loop(0, CHUNK // 16)
        def _vec(i):
            d16 = dstv[ch, pl.ds(i * 16, 16)]
            rd = plsc.load_gather(rden_v, [d16])
            al = exv[ch, pl.ds(i * 16, 16)] * rd
            for l in range(16):
                j = i * 16 + l
                acc = vrows[j, pl.ds(0, 16)] * hrows[j, pl.ds(0, 16)]
                for q in range(1, HID // 16):
                    acc = acc + vrows[j, pl.ds(q * 16, 16)] * hrows[j, pl.ds(q * 16, 16)]
                grad = jnp.sum(acc) * inv_n
                b = al[l] - grad
                one = jnp.float32(1.0)
                bn = jnp.where(b > 0, one, jnp.where(b < 0, -one, 0.0))
                obuf[j, pl.ds(0, H2W)] = vrows[j, pl.ds(HID, H2W)] * bn

    def pipeline(nch):
        g_start(0, v_a, h_a, gva, gha)
        g_start(1, v_b, h_b, gvb, ghb)

        @pl.loop(0, nch // 2)
        def _pipe(i):
            ch0 = i * 2
            ch1 = ch0 + 1
            g_wait(ch0, v_a, h_a, gva, gha)

            @pl.when(i > 0)
            def _():
                s_wait(ch0, o_a, osa)
            compute(ch0, v_a, h_a, o_a)
            s_start(ch0, o_a, osa)

            @pl.when(i < nch // 2 - 1)
            def _():
                g_start(ch0 + 2, v_a, h_a, gva, gha)

            g_wait(ch1, v_b, h_b, gvb, ghb)

            @pl.when(i > 0)
            def _():
                s_wait(ch1, o_b, osb)
            compute(ch1, v_b, h_b, o_b)
            s_start(ch1, o_b, osb)

            @pl.when(i < nch // 2 - 1)
            def _():
                g_start(ch1 + 2, v_b, h_b, gvb, ghb)

        s_wait(0, o_a, osa)
        s_wait(1, o_b, osb)

    @pl.when(c == FAST_CORE)
    def _():
        pipeline(NCH3_F)

    @pl.when(c != FAST_CORE)
    def _():
        pipeline(NCH3_S)

    plsc.subcore_barrier()
    for q in range(TSLICE // CHUNK):
        buf = o_a if q % 2 == 0 else o_b
        pltpu.sync_copy(o2_sh.at[pl.ds(s * TSLICE + q * CHUNK, CHUNK)], buf)
        pltpu.sync_copy(buf, o2_h.at[wid * (TSLICE // CHUNK) + q])


def _run_sc3(src3, dst3, ex3, rden, vh2p, h1p):
    kern = pl.kernel(
        _sc3_body,
        out_type=[
            jax.ShapeDtypeStruct((NWORK * TSLICE // CHUNK, CHUNK, H2W),
                                 jnp.float32),
        ],
        mesh=plsc.VectorSubcoreMesh(core_axis_name="core",
                                    subcore_axis_name="subcore"),
        scratch_types=[
            pltpu.VMEM((ESZ,), jnp.float32),
            pltpu.VMEM((NCH3_F, CHUNK), jnp.int32),
            pltpu.VMEM((NCH3_F, CHUNK), jnp.int32),
            pltpu.VMEM((NCH3_F, CHUNK), jnp.float32),
            pltpu.VMEM((CHUNK, TW), jnp.float32),
            pltpu.VMEM((CHUNK, TW), jnp.float32),
            pltpu.VMEM((CHUNK, TW), jnp.float32),
            pltpu.VMEM((CHUNK, TW), jnp.float32),
            pltpu.VMEM((CHUNK, H2W), jnp.float32),
            pltpu.VMEM((CHUNK, H2W), jnp.float32),
            pltpu.SemaphoreType.DMA,
            pltpu.SemaphoreType.DMA,
            pltpu.SemaphoreType.DMA,
            pltpu.SemaphoreType.DMA,
            pltpu.SemaphoreType.DMA,
            pltpu.SemaphoreType.DMA,
            pltpu.VMEM_SHARED((NP_, H2W), jnp.float32),
        ],
        compiler_params=_sc_compiler_params(),
    )
    return kern(src3, dst3, ex3, rden, vh2p, h1p)


def _p3_body(o2_ref, out_ref):
    out_ref[...] = o2_ref[0] + o2_ref[1]


def _run_p3(o22):
    return pl.pallas_call(
        _p3_body,
        grid=(GRID,),
        in_specs=[pl.BlockSpec((2, NB, H2W), lambda j: (0, j, 0))],
        out_specs=pl.BlockSpec((NB, H2W), lambda j: (j, 0)),
        out_shape=jax.ShapeDtypeStruct((NP_, H2W), jnp.float32),
    )(o22)


def kernel(x, edge_index, y, C_b_prime, Q, W1, b1, a1_src, a1_dst, W2, b2):
    x_p = jnp.pad(x, ((0, NP_ - N), (0, 0)))
    src = jnp.pad(edge_index[0], (0, EP_ - E), constant_values=0)
    dst = jnp.pad(edge_index[1], (0, EP_ - E), constant_values=N)
    src3 = src.reshape(NCHTOT, CHUNK)
    dst3 = dst.reshape(NCHTOT, CHUNK)
    y2 = jnp.pad(y, (0, NP_ - N), constant_values=K + 1).reshape(1, NP_)
    b1r = b1.reshape(1, HID)
    a1s = a1_src.reshape(HID, 1)
    a1d = a1_dst.reshape(HID, 1)
    W2p = jnp.pad(W2, ((0, 0), (0, H2W - OUT)))
    b2p = jnp.pad(b2, (0, H2W - OUT)).reshape(1, H2W)
    h1p, es, ed, lps = _run_p0(x_p, W1, b1r, a1s, a1d, C_b_prime)
    ex3, acc2 = _run_sc1(es.reshape(NP_), ed.reshape(NP_), src3, dst3, h1p)
    z0p, h2p, rden, ca = _run_p2a(acc2.reshape(2, NP_, TW), y2, W2p, b2p)
    phip, vh2p = _run_p2b(z0p, ca, y2, lps, Q, h2p)
    (o22,) = _run_sc3(src3, dst3, ex3, rden.reshape(NP_), vh2p, h1p)
    out2p = _run_p3(o22.reshape(2, NP_, H2W))

    return (out2p[:N, :OUT], h2p[:N, :OUT], z0p[:N], phip[:N])


# speedup vs baseline: 1.0912x; 1.0912x over previous
import dataclasses
import functools

import jax
import jax.numpy as jnp
from jax import lax
from jax.experimental import pallas as pl
from jax.experimental.pallas import tpu as pltpu
from jax.experimental.pallas import tpu_sc as plsc

N = 10000
E = 160000
D_IN = 128
HID = 64
OUT = 7
K = 7

NP_ = 10240
EP_ = 163840
NWORK = 32
CHUNK = 128
NCHUNK = (EP_ // NWORK) // CHUNK
TSLICE = NP_ // 16
NB = 512
TW = 80

def _sc_compiler_params():
    cp = pltpu.CompilerParams()
    if "needs_layout_passes" in pltpu.CompilerParams.__dataclass_fields__:
        cp = dataclasses.replace(cp, needs_layout_passes=False)
    cp = dataclasses.replace(cp, use_tc_tiling_on_sc=False)
    return cp

GRID = NP_ // NB
H2W = 16


def _p0_body(x_ref, w1_ref, b1_ref, a1s_ref, a1d_ref, cb_ref,
             h1_ref, es_ref, ed_ref, lps_ref):
    xb = x_ref[...]
    h1 = jnp.dot(xb, w1_ref[...], preferred_element_type=jnp.float32)
    h1 = h1 + b1_ref[...]
    h1_ref[...] = jnp.concatenate(
        [h1, jnp.ones((NB, 1), jnp.float32),
         jnp.zeros((NB, TW - HID - 1), jnp.float32)], axis=1)
    es_ref[0, :] = jnp.dot(h1, a1s_ref[...], preferred_element_type=jnp.float32)[:, 0]
    ed_ref[0, :] = jnp.dot(h1, a1d_ref[...], preferred_element_type=jnp.float32)[:, 0]
    ssum = jnp.zeros((NB,), jnp.float32)
    slog = jnp.zeros((NB,), jnp.float32)
    for k in range(K):
        d = xb - cb_ref[k, :][None, :]
        dist = jnp.sqrt(jnp.sum(d * d, axis=1) + 1e-12)
        sc = jnp.exp(-dist) + 1e-10
        ssum = ssum + sc
        slog = slog + jnp.log(sc)
    lps_ref[0, :] = slog - K * jnp.log(ssum)


def _run_p0(x_p, W1, b1r, a1s, a1d, C_b):
    return pl.pallas_call(
        _p0_body,
        grid=(GRID,),
        in_specs=[
            pl.BlockSpec((NB, D_IN), lambda j: (j, 0)),
            pl.BlockSpec((D_IN, HID), lambda j: (0, 0)),
            pl.BlockSpec((1, HID), lambda j: (0, 0)),
            pl.BlockSpec((HID, 1), lambda j: (0, 0)),
            pl.BlockSpec((HID, 1), lambda j: (0, 0)),
            pl.BlockSpec((K, D_IN), lambda j: (0, 0)),
        ],
        out_specs=[
            pl.BlockSpec((NB, TW), lambda j: (j, 0)),
            pl.BlockSpec((1, NB), lambda j: (0, j)),
            pl.BlockSpec((1, NB), lambda j: (0, j)),
            pl.BlockSpec((1, NB), lambda j: (0, j)),
        ],
        out_shape=[
            jax.ShapeDtypeStruct((NP_, TW), jnp.float32),
            jax.ShapeDtypeStruct((1, NP_), jnp.float32),
            jax.ShapeDtypeStruct((1, NP_), jnp.float32),
            jax.ShapeDtypeStruct((1, NP_), jnp.float32),
        ],
    )(x_p, W1, b1r, a1s, a1d, C_b)


def _sc1_body(es_h, ed_h, src_h, dst_h, h1_h,
              ex_h, acc_h,
              es_v, ed_v, srcv, dstv, exv, rows_a, rows_b,
              r80_a, r80_b,
              gsa, gsb, ssa, ssb,
              acc_sh):
    c = lax.axis_index("core")
    s = lax.axis_index("subcore")
    wid = c * 16 + s

    z16v = jnp.zeros((16,), jnp.float32)

    @pl.loop(0, CHUNK)
    def _zr(r):
        for q in range(TW // 16):
            r80_a[r, pl.ds(q * 16, 16)] = z16v

    for q in range(TSLICE // CHUNK):
        pltpu.sync_copy(r80_a, acc_sh.at[pl.ds(s * TSLICE + q * CHUNK, CHUNK)])
    pltpu.sync_copy(es_h, es_v)
    pltpu.sync_copy(ed_h, ed_v)
    pltpu.sync_copy(src_h.at[wid], srcv)
    pltpu.sync_copy(dst_h.at[wid], dstv)
    plsc.subcore_barrier()

    @pl.loop(0, NCHUNK)
    def _exch(ch):
        @pl.loop(0, CHUNK // 16)
        def _vec(i):
            s16 = srcv[ch, pl.ds(i * 16, 16)]
            d16 = dstv[ch, pl.ds(i * 16, 16)]
            ev = plsc.load_gather(es_v, [s16])
            dv = plsc.load_gather(ed_v, [d16])
            e = ev + dv
            e = jnp.where(e > 0, e, 0.2 * e)
            exv[ch, pl.ds(i * 16, 16)] = jnp.exp(e)

    def g_start(ch, buf, sem):
        pltpu.make_async_copy(h1_h.at[srcv.at[ch]], buf, sem).start()

    def g_wait(ch, buf, sem):
        pltpu.make_async_copy(h1_h.at[srcv.at[ch]], buf, sem).wait()

    def s_start(ch, obuf, sem):
        pltpu.make_async_copy(obuf, acc_sh.at[dstv.at[ch]], sem).start(add=True)

    def s_wait(ch, obuf, sem):
        pltpu.make_async_copy(obuf, acc_sh.at[dstv.at[ch]], sem).wait()

    def scale(ch, buf, obuf):
        @pl.loop(0, CHUNK // 16)
        def _scale(i):
            exvec = exv[ch, pl.ds(i * 16, 16)]
            for l in range(16):
                j = i * 16 + l
                sc = exvec[l]
                for q in range(TW // 16):
                    obuf[j, pl.ds(q * 16, 16)] = buf[j, pl.ds(q * 16, 16)] * sc

    g_start(0, rows_a, gsa)
    g_start(1, rows_b, gsb)

    @pl.loop(0, NCHUNK // 2)
    def _pipe(i):
        ch0 = i * 2
        ch1 = ch0 + 1
        g_wait(ch0, rows_a, gsa)

        @pl.when(i > 0)
        def _():
            s_wait(ch0, r80_a, ssa)
        scale(ch0, rows_a, r80_a)
        s_start(ch0, r80_a, ssa)

        @pl.when(i < NCHUNK // 2 - 1)
        def _():
            g_start(ch0 + 2, rows_a, gsa)

        g_wait(ch1, rows_b, gsb)

        @pl.when(i > 0)
        def _():
            s_wait(ch1, r80_b, ssb)
        scale(ch1, rows_b, r80_b)
        s_start(ch1, r80_b, ssb)

        @pl.when(i < NCHUNK // 2 - 1)
        def _():
            g_start(ch1 + 2, rows_b, gsb)

    s_wait(0, r80_a, ssa)
    s_wait(1, r80_b, ssb)

    pltpu.sync_copy(exv, ex_h.at[wid])
    plsc.subcore_barrier()
    for q in range(TSLICE // CHUNK):
        buf = r80_a if q % 2 == 0 else r80_b
        pltpu.sync_copy(acc_sh.at[pl.ds(s * TSLICE + q * CHUNK, CHUNK)], buf)
        pltpu.sync_copy(buf, acc_h.at[wid * (TSLICE // CHUNK) + q])


def _run_sc1(es, ed, src3, dst3, h1p):
    kern = pl.kernel(
        _sc1_body,
        out_type=[
            jax.ShapeDtypeStruct((NWORK, NCHUNK, CHUNK), jnp.float32),
            jax.ShapeDtypeStruct((NWORK * TSLICE // CHUNK, CHUNK, TW),
                                 jnp.float32),
        ],
        mesh=plsc.VectorSubcoreMesh(core_axis_name="core",
                                    subcore_axis_name="subcore"),
        scratch_types=[
            pltpu.VMEM((NP_,), jnp.float32),
            pltpu.VMEM((NP_,), jnp.float32),
            pltpu.VMEM((NCHUNK, CHUNK), jnp.int32),
            pltpu.VMEM((NCHUNK, CHUNK), jnp.int32),
            pltpu.VMEM((NCHUNK, CHUNK), jnp.float32),
            pltpu.VMEM((CHUNK, TW), jnp.float32),
            pltpu.VMEM((CHUNK, TW), jnp.float32),
            pltpu.VMEM((CHUNK, TW), jnp.float32),
            pltpu.VMEM((CHUNK, TW), jnp.float32),
            pltpu.SemaphoreType.DMA,
            pltpu.SemaphoreType.DMA,
            pltpu.SemaphoreType.DMA,
            pltpu.SemaphoreType.DMA,
            pltpu.VMEM_SHARED((NP_, TW), jnp.float32),
        ],
        compiler_params=_sc_compiler_params(),
    )
    return kern(es, ed, src3, dst3, h1p)


def _p2a_body(acc_ref, y_ref, w2_ref, b2_ref,
              z0_ref, h2_ref, rden_ref, ca_ref, sums_ref, cnt_ref):
    j = pl.program_id(0)

    @pl.when(j == 0)
    def _():
        sums_ref[...] = jnp.zeros_like(sums_ref)
        cnt_ref[...] = jnp.zeros_like(cnt_ref)

    a = acc_ref[0] + acc_ref[1]
    rden = 1.0 / (a[:, HID] + 1e-16)
    rden_ref[0, :] = rden
    z0 = a[:, :HID] * rden[:, None]
    z0_ref[...] = z0
    x1 = jnp.where(z0 > 0, z0, jnp.exp(jnp.minimum(z0, 0.0)) - 1.0)
    h2_ref[...] = jnp.dot(x1, w2_ref[...], preferred_element_type=jnp.float32) + b2_ref[...]
    yb = y_ref[0, :]
    y1h = (lax.broadcasted_iota(jnp.int32, (NB, 8), 1) == yb[:, None]).astype(jnp.float32)
    sums_ref[...] += jax.lax.dot_general(
        y1h, z0, (((0,), (0,)), ((), ())), preferred_element_type=jnp.float32)
    cnt_ref[...] += jnp.sum(y1h, axis=0, keepdims=True)
    ca_ref[...] = sums_ref[...] / jnp.maximum(cnt_ref[0, :], 1.0)[:, None]


def _run_p2a(acc2, y2, W2p, b2p):
    return pl.pallas_call(
        _p2a_body,
        grid=(GRID,),
        in_specs=[
            pl.BlockSpec((2, NB, TW), lambda j: (0, j, 0)),
            pl.BlockSpec((1, NB), lambda j: (0, j)),
            pl.BlockSpec((HID, H2W), lambda j: (0, 0)),
            pl.BlockSpec((1, H2W), lambda j: (0, 0)),
        ],
        out_specs=[
            pl.BlockSpec((NB, HID), lambda j: (j, 0)),
            pl.BlockSpec((NB, H2W), lambda j: (j, 0)),
            pl.BlockSpec((1, NB), lambda j: (0, j)),
            pl.BlockSpec((8, HID), lambda j: (0, 0)),
        ],
        out_shape=[
            jax.ShapeDtypeStruct((NP_, HID), jnp.float32),
            jax.ShapeDtypeStruct((NP_, H2W), jnp.float32),
            jax.ShapeDtypeStruct((1, NP_), jnp.float32),
            jax.ShapeDtypeStruct((8, HID), jnp.float32),
        ],
        scratch_shapes=[
            pltpu.VMEM((8, HID), jnp.float32),
            pltpu.VMEM((1, 8), jnp.float32),
        ],
    )(acc2, y2, W2p, b2p)


def _p2b_body(z0_ref, ca_ref, y_ref, lps_ref, q_ref, h2_ref, phi_ref, v_ref):
    zb = z0_ref[...]
    dists = []
    for k in range(K):
        dk = zb - ca_ref[k, :][None, :]
        dists.append(jnp.sqrt(jnp.sum(dk * dk, axis=1) + 1e-12))
    dist = jnp.stack(dists, axis=1)
    sc = jnp.exp(-dist) + 1e-10
    phi = sc / jnp.sum(sc, axis=1, keepdims=True)
    phi_ref[...] = phi
    yb = y_ref[0, :]
    y1h = (lax.broadcasted_iota(jnp.int32, (NB, 8), 1) == yb[:, None]).astype(jnp.float32)
    lq = jnp.log(q_ref[...])
    lq8 = jnp.concatenate([lq, jnp.zeros((1, K), jnp.float32)], axis=0)
    diff_b = phi * lps_ref[0, :][:, None] - jnp.dot(
        y1h, lq8, preferred_element_type=jnp.float32)
    dp = diff_b * phi
    dpsum = jnp.sum(dp, axis=1, keepdims=True)
    w = jnp.zeros((NB, HID), jnp.float32)
    t = jnp.zeros((NB, HID), jnp.float32)
    for k in range(K):
        dk = zb - ca_ref[k, :][None, :]
        dn = dk / dist[:, k][:, None]
        w = w + phi[:, k][:, None] * dn
        t = t + dp[:, k][:, None] * dn
    v = -(t - dpsum * w)
    v_ref[...] = jnp.concatenate([v, h2_ref[...]], axis=1)


def _run_p2b(z0p, ca, y2, lps, Q, h2p):
    return pl.pallas_call(
        _p2b_body,
        grid=(GRID,),
        in_specs=[
            pl.BlockSpec((NB, HID), lambda j: (j, 0)),
            pl.BlockSpec((8, HID), lambda j: (0, 0)),
            pl.BlockSpec((1, NB), lambda j: (0, j)),
            pl.BlockSpec((1, NB), lambda j: (0, j)),
            pl.BlockSpec((K, K), lambda j: (0, 0)),
            pl.BlockSpec((NB, H2W), lambda j: (j, 0)),
        ],
        out_specs=[
            pl.BlockSpec((NB, K), lambda j: (j, 0)),
            pl.BlockSpec((NB, TW), lambda j: (j, 0)),
        ],
        out_shape=[
            jax.ShapeDtypeStruct((NP_, K), jnp.float32),
            jax.ShapeDtypeStruct((NP_, TW), jnp.float32),
        ],
    )(z0p, ca, y2, lps, Q, h2p)


def _sc3_body(src_h, dst_h, ex_h, rden_h, vh2_h, h1_h,
              o2_h,
              rden_v, srcv, dstv, exv, v_a, v_b, h_a, h_b, o_a, o_b,
              gva, gvb, gha, ghb, osa, osb,
              o2_sh):
    c = lax.axis_index("core")
    s = lax.axis_index("subcore")
    wid = c * 16 + s

    z16v = jnp.zeros((16,), jnp.float32)

    @pl.loop(0, CHUNK)
    def _zr(r):
        o_a[r, pl.ds(0, H2W)] = z16v

    for q in range(TSLICE // CHUNK):
        pltpu.sync_copy(o_a, o2_sh.at[pl.ds(s * TSLICE + q * CHUNK, CHUNK)])
    pltpu.sync_copy(rden_h, rden_v)
    pltpu.sync_copy(src_h.at[wid], srcv)
    pltpu.sync_copy(dst_h.at[wid], dstv)
    pltpu.sync_copy(ex_h.at[wid], exv)
    plsc.subcore_barrier()

    inv_n = jnp.float32(1.0 / N)

    def g_start(ch, vbuf, hbuf, vsem, hsem):
        pltpu.make_async_copy(vh2_h.at[srcv.at[ch]], vbuf, vsem).start()
        pltpu.make_async_copy(h1_h.at[dstv.at[ch]], hbuf, hsem).start()

    def g_wait(ch, vbuf, hbuf, vsem, hsem):
        pltpu.make_async_copy(vh2_h.at[srcv.at[ch]], vbuf, vsem).wait()
        pltpu.make_async_copy(h1_h.at[dstv.at[ch]], hbuf, hsem).wait()

    def s_start(ch, obuf, sem):
        pltpu.make_async_copy(obuf, o2_sh.at[dstv.at[ch]], sem).start(add=True)

    def s_wait(ch, obuf, sem):
        pltpu.make_async_copy(obuf, o2_sh.at[dstv.at[ch]], sem).wait()

    def compute(ch, vrows, hrows, obuf):
        @pl.loop(0, CHUNK // 16)
        def _vec(i):
            d16 = dstv[ch, pl.ds(i * 16, 16)]
            rd = plsc.load_gather(rden_v, [d16])
            al = exv[ch, pl.ds(i * 16, 16)] * rd
            for l in range(16):
                j = i * 16 + l
                acc = vrows[j, pl.ds(0, 16)] * hrows[j, pl.ds(0, 16)]
                for q in range(1, HID // 16):
                    acc = acc + vrows[j, pl.ds(q * 16, 16)] * hrows[j, pl.ds(q * 16, 16)]
                grad = jnp.sum(acc) * inv_n
                b = al[l] - grad
                one = jnp.float32(1.0)
                bn = jnp.where(b > 0, one, jnp.where(b < 0, -one, 0.0))
                obuf[j, pl.ds(0, H2W)] = vrows[j, pl.ds(HID, H2W)] * bn

    g_start(0, v_a, h_a, gva, gha)
    g_start(1, v_b, h_b, gvb, ghb)

    @pl.loop(0, NCHUNK // 2)
    def _pipe(i):
        ch0 = i * 2
        ch1 = ch0 + 1
        g_wait(ch0, v_a, h_a, gva, gha)

        @pl.when(i > 0)
        def _():
            s_wait(ch0, o_a, osa)
        compute(ch0, v_a, h_a, o_a)
        s_start(ch0, o_a, osa)

        @pl.when(i < NCHUNK // 2 - 1)
        def _():
            g_start(ch0 + 2, v_a, h_a, gva, gha)

        g_wait(ch1, v_b, h_b, gvb, ghb)

        @pl.when(i > 0)
        def _():
            s_wait(ch1, o_b, osb)
        compute(ch1, v_b, h_b, o_b)
        s_start(ch1, o_b, osb)

        @pl.when(i < NCHUNK // 2 - 1)
        def _():
            g_start(ch1 + 2, v_b, h_b, gvb, ghb)

    s_wait(0, o_a, osa)
    s_wait(1, o_b, osb)

    plsc.subcore_barrier()
    for q in range(TSLICE // CHUNK):
        buf = o_a if q % 2 == 0 else o_b
        pltpu.sync_copy(o2_sh.at[pl.ds(s * TSLICE + q * CHUNK, CHUNK)], buf)
        pltpu.sync_copy(buf, o2_h.at[wid * (TSLICE // CHUNK) + q])


def _run_sc3(src3, dst3, ex3, rden, vh2p, h1p):
    kern = pl.kernel(
        _sc3_body,
        out_type=[
            jax.ShapeDtypeStruct((NWORK * TSLICE // CHUNK, CHUNK, H2W),
                                 jnp.float32),
        ],
        mesh=plsc.VectorSubcoreMesh(core_axis_name="core",
                                    subcore_axis_name="subcore"),
        scratch_types=[
            pltpu.VMEM((NP_,), jnp.float32),
            pltpu.VMEM((NCHUNK, CHUNK), jnp.int32),
            pltpu.VMEM((NCHUNK, CHUNK), jnp.int32),
            pltpu.VMEM((NCHUNK, CHUNK), jnp.float32),
            pltpu.VMEM((CHUNK, TW), jnp.float32),
            pltpu.VMEM((CHUNK, TW), jnp.float32),
            pltpu.VMEM((CHUNK, TW), jnp.float32),
            pltpu.VMEM((CHUNK, TW), jnp.float32),
            pltpu.VMEM((CHUNK, H2W), jnp.float32),
            pltpu.VMEM((CHUNK, H2W), jnp.float32),
            pltpu.SemaphoreType.DMA,
            pltpu.SemaphoreType.DMA,
            pltpu.SemaphoreType.DMA,
            pltpu.SemaphoreType.DMA,
            pltpu.SemaphoreType.DMA,
            pltpu.SemaphoreType.DMA,
            pltpu.VMEM_SHARED((NP_, H2W), jnp.float32),
        ],
        compiler_params=_sc_compiler_params(),
    )
    return kern(src3, dst3, ex3, rden, vh2p, h1p)


def _p3_body(o2_ref, out_ref):
    out_ref[...] = o2_ref[0] + o2_ref[1]


def _run_p3(o22):
    return pl.pallas_call(
        _p3_body,
        grid=(GRID,),
        in_specs=[pl.BlockSpec((2, NB, H2W), lambda j: (0, j, 0))],
        out_specs=pl.BlockSpec((NB, H2W), lambda j: (j, 0)),
        out_shape=jax.ShapeDtypeStruct((NP_, H2W), jnp.float32),
    )(o22)


def kernel(x, edge_index, y, C_b_prime, Q, W1, b1, a1_src, a1_dst, W2, b2):
    x_p = jnp.pad(x, ((0, NP_ - N), (0, 0)))
    src = jnp.pad(edge_index[0], (0, EP_ - E), constant_values=NP_ - 1)
    dst = jnp.pad(edge_index[1], (0, EP_ - E), constant_values=NP_ - 1)
    src3 = src.reshape(NWORK, NCHUNK, CHUNK)
    dst3 = dst.reshape(NWORK, NCHUNK, CHUNK)
    y2 = jnp.pad(y, (0, NP_ - N), constant_values=K + 1).reshape(1, NP_)
    b1r = b1.reshape(1, HID)
    a1s = a1_src.reshape(HID, 1)
    a1d = a1_dst.reshape(HID, 1)
    W2p = jnp.pad(W2, ((0, 0), (0, H2W - OUT)))
    b2p = jnp.pad(b2, (0, H2W - OUT)).reshape(1, H2W)
    h1p, es, ed, lps = _run_p0(x_p, W1, b1r, a1s, a1d, C_b_prime)
    ex3, acc2 = _run_sc1(es.reshape(NP_), ed.reshape(NP_), src3, dst3, h1p)
    z0p, h2p, rden, ca = _run_p2a(acc2.reshape(2, NP_, TW), y2, W2p, b2p)
    phip, vh2p = _run_p2b(z0p, ca, y2, lps, Q, h2p)
    (o22,) = _run_sc3(src3, dst3, ex3, rden.reshape(NP_), vh2p, h1p)
    out2p = _run_p3(o22.reshape(2, NP_, H2W))

    return (out2p[:N, :OUT], h2p[:N, :OUT], z0p[:N], phip[:N])
